# Initial kernel scaffold; baseline (speedup 1.0000x reference)
import jax
import jax.numpy as jnp
from jax.experimental import pallas as pl


def kernel(x, edge_index, batch, W0, b0, W1, b1, W2, b2, Wc1, bc1, Wc2, bc2):
    raise NotImplementedError("write your pallas kernel here")


# speedup vs baseline: 8.6667x; 8.6667x over previous
import functools

import jax
import jax.numpy as jnp
from jax import lax
from jax.experimental import pallas as pl
from jax.experimental.pallas import tpu as pltpu
from jax.experimental.pallas import tpu_sc as plsc

N = 10000
NP = 10240
D = 128
G = 64
OUT = 10
E = 320000
CH = 128
NCORE = 2
NSUB = 16
NW = NCORE * NSUB
NCH = 79
EPW = NCH * CH
EP = EPW * NW
RPS = NP // NSUB // CH

_f32 = jnp.float32
_HIGH = jax.lax.Precision.HIGHEST


@functools.cache
def _sc_kernels():
    mesh = plsc.VectorSubcoreMesh(core_axis_name="c", subcore_axis_name="s")

    @functools.partial(
        pl.kernel,
        out_type=jax.ShapeDtypeStruct((NCORE * NP, D), _f32),
        mesh=mesh,
        scratch_types=[
            pltpu.VMEM((NCH, 1, CH), jnp.int32),
            pltpu.VMEM((CH, D), _f32),
            pltpu.VMEM((CH, D), _f32),
            pltpu.VMEM_SHARED((NP, D), _f32),
            pltpu.SemaphoreType.DMA,
        ],
    )
    def deg_hist(dst_hbm, ones_hbm, zeros_hbm, out_hbm,
                 idx_da, obuf, zbuf, acc, sem):
        c = lax.axis_index("c")
        s = lax.axis_index("s")
        w = c * NSUB + s
        pltpu.sync_copy(ones_hbm, obuf)
        pltpu.sync_copy(zeros_hbm, zbuf)
        pltpu.sync_copy(dst_hbm.at[pl.ds(w * NCH, NCH)], idx_da)

        @pl.loop(0, RPS)
        def _zero(k):
            pltpu.sync_copy(zbuf, acc.at[pl.ds((s * RPS + k) * CH, CH)])

        plsc.subcore_barrier()

        @pl.loop(0, NCH - 7, step=8)
        def _edges(k):
            for b in range(8):
                pltpu.async_copy(obuf, acc.at[idx_da.at[k + b].at[0]], sem,
                                 add=True)
            for b in range(8):
                pltpu.make_async_copy(
                    obuf, acc.at[idx_da.at[k + b].at[0]], sem).wait()

        @pl.loop(NCH - NCH % 8, NCH)
        def _tail(k):
            pltpu.sync_copy(obuf, acc.at[idx_da.at[k].at[0]], add=True)

        plsc.subcore_barrier()

        @pl.loop(0, RPS)
        def _wb(k):
            r = (s * RPS + k) * CH
            pltpu.sync_copy(acc.at[pl.ds(r, CH)],
                            out_hbm.at[pl.ds(c * NP + r, CH)])

    @functools.partial(
        pl.kernel,
        out_type=jax.ShapeDtypeStruct((NCORE * NP, D), _f32),
        mesh=mesh,
        scratch_types=[
            pltpu.VMEM((1, CH), jnp.int32),
            pltpu.VMEM((1, CH), jnp.int32),
            pltpu.VMEM((CH, D), _f32),
            pltpu.VMEM((CH, D), _f32),
            pltpu.VMEM_SHARED((NP, D), _f32),
        ],
    )
    def edge_pass(y_hbm, src_hbm, dst_hbm, zeros_hbm, out_hbm,
                  idx_s, idx_d, gbuf, zbuf, acc):
        c = lax.axis_index("c")
        s = lax.axis_index("s")
        pltpu.sync_copy(zeros_hbm, zbuf)

        @pl.loop(0, RPS)
        def _zero(k):
            pltpu.sync_copy(zbuf, acc.at[pl.ds((s * RPS + k) * CH, CH)])

        plsc.subcore_barrier()
        base = (c * NSUB + s) * NCH

        @pl.loop(0, NCH)
        def _edges(k):
            pltpu.sync_copy(src_hbm.at[base + k], idx_s)
            pltpu.sync_copy(dst_hbm.at[base + k], idx_d)
            pltpu.sync_copy(y_hbm.at[idx_s.at[0]], gbuf)
            pltpu.sync_copy(gbuf, acc.at[idx_d.at[0]], add=True)

        plsc.subcore_barrier()

        @pl.loop(0, RPS)
        def _wb(k):
            r = (s * RPS + k) * CH
            pltpu.sync_copy(acc.at[pl.ds(r, CH)],
                            out_hbm.at[pl.ds(c * NP + r, CH)])

    return deg_hist, edge_pass


def _dinv(deg_ref):
    d = deg_ref[0:NP, 0:1] + deg_ref[NP:2 * NP, 0:1]
    return lax.rsqrt(1.0 + d)


def _lin0_body(x_ref, w_ref, deg_ref, y_ref):
    xw = jnp.dot(x_ref[...], w_ref[...], precision=_HIGH,
                 preferred_element_type=_f32)
    y_ref[...] = _dinv(deg_ref) * xw


_lin0 = pl.pallas_call(
    _lin0_body, out_shape=jax.ShapeDtypeStruct((NP, D), _f32))


def _lin_mid_body(acc_ref, y_ref, deg_ref, b_ref, w_ref, o_ref):
    dinv = _dinv(deg_ref)
    accs = acc_ref[0:NP, :] + acc_ref[NP:2 * NP, :]
    h = jnp.maximum(dinv * (accs + y_ref[...]) + b_ref[...], 0.0)
    o_ref[...] = dinv * jnp.dot(h, w_ref[...], precision=_HIGH,
                                preferred_element_type=_f32)


_lin_mid = pl.pallas_call(
    _lin_mid_body, out_shape=jax.ShapeDtypeStruct((NP, D), _f32))


def _final_body(acc_ref, y_ref, deg_ref, b_ref, batch_ref, batch_c_ref,
                wc1_ref, bc1_ref, wc2_ref, bc2_ref, o_ref, h_ref, m_ref):
    dinv = _dinv(deg_ref)
    accs = acc_ref[0:NP, :] + acc_ref[NP:2 * NP, :]
    h = dinv * (accs + y_ref[...]) + b_ref[...]
    h_ref[...] = h

    gids = lax.broadcasted_iota(jnp.int32, (G, 1), 0)
    mask = (gids == batch_ref[...]).astype(_f32)
    cnt = jnp.sum(mask, axis=1, keepdims=True)
    sums = jnp.dot(mask, h, precision=_HIGH, preferred_element_type=_f32)
    mean = sums / jnp.maximum(cnt, 1.0)

    batch_c = batch_c_ref[...]

    def mx_step(g, _):
        cand = jnp.where(batch_c == g, h_ref[...], -jnp.inf)
        m_ref[pl.ds(g, 1), :] = jnp.max(cand, axis=0, keepdims=True)
        return 0

    lax.fori_loop(0, G, mx_step, 0)
    m = m_ref[...]
    mx = jnp.where(jnp.isfinite(m), m, 0.0)

    p = jnp.concatenate([mean, mx], axis=1)
    z = jnp.maximum(jnp.dot(p, wc1_ref[...], precision=_HIGH,
                            preferred_element_type=_f32) + bc1_ref[...], 0.0)
    o_ref[...] = jnp.dot(z, wc2_ref[...], precision=_HIGH,
                         preferred_element_type=_f32) + bc2_ref[...]


_final = pl.pallas_call(
    _final_body, out_shape=jax.ShapeDtypeStruct((G, D), _f32),
    scratch_shapes=[pltpu.VMEM((NP, D), _f32), pltpu.VMEM((G, D), _f32)])


def kernel(x, edge_index, batch, W0, b0, W1, b1, W2, b2, Wc1, bc1, Wc2, bc2):
    i32 = jnp.int32
    src_p = jnp.concatenate([edge_index[0].astype(i32),
                             jnp.zeros((EP - E,), i32)]).reshape(NW * NCH, 1, CH)
    dst_p = jnp.concatenate([edge_index[1].astype(i32),
                             jnp.full((EP - E,), N, i32)]).reshape(NW * NCH, 1, CH)
    x_p = jnp.pad(x, ((0, NP - N), (0, 0)))
    batch_p = jnp.pad(batch.astype(i32), (0, NP - N), constant_values=G)
    batch_t = batch_p.reshape(1, NP)
    batch_c = batch_p.reshape(NP, 1)
    onesD = jnp.ones((CH, D), _f32)
    zerosD = jnp.zeros((CH, D), _f32)
    wc2_p = jnp.pad(Wc2, ((0, 0), (0, D - OUT)))
    bc2_p = jnp.pad(bc2, (0, D - OUT)).reshape(1, D)

    deg_hist, edge_pass = _sc_kernels()
    deg = deg_hist(dst_p, onesD, zerosD)
    y0 = _lin0(x_p, W0, deg)
    acc0 = edge_pass(y0, src_p, dst_p, zerosD)
    y1 = _lin_mid(acc0, y0, deg, b0.reshape(1, D), W1)
    acc1 = edge_pass(y1, src_p, dst_p, zerosD)
    y2 = _lin_mid(acc1, y1, deg, b1.reshape(1, D), W2)
    acc2 = edge_pass(y2, src_p, dst_p, zerosD)
    out = _final(acc2, y2, deg, b2.reshape(1, D), batch_t, batch_c,
                 Wc1, bc1.reshape(1, D), wc2_p, bc2_p)
    return out[:, :OUT]


# speedup vs baseline: 9.9519x; 1.1483x over previous
import functools

import jax
import jax.numpy as jnp
from jax import lax
from jax.experimental import pallas as pl
from jax.experimental.pallas import tpu as pltpu
from jax.experimental.pallas import tpu_sc as plsc

N = 10000
NP = 10240
D = 128
G = 64
OUT = 10
E = 320000
CH = 128
NCORE = 2
NSUB = 16
NW = NCORE * NSUB
NCH = 79
EPW = NCH * CH
EP = EPW * NW
RPS = NP // NSUB // CH

_f32 = jnp.float32
_HIGH = jax.lax.Precision.HIGHEST


@functools.cache
def _sc_kernels():
    mesh = plsc.VectorSubcoreMesh(core_axis_name="c", subcore_axis_name="s")

    @functools.partial(
        pl.kernel,
        out_type=jax.ShapeDtypeStruct((NCORE * NP, D), _f32),
        mesh=mesh,
        scratch_types=[
            pltpu.VMEM((NCH, 1, CH), jnp.int32),
            pltpu.VMEM((CH, D), _f32),
            pltpu.VMEM((CH, D), _f32),
            pltpu.VMEM_SHARED((NP, D), _f32),
            pltpu.SemaphoreType.DMA,
        ],
    )
    def deg_hist(dst_hbm, ones_hbm, zeros_hbm, out_hbm,
                 idx_da, obuf, zbuf, acc, sem):
        c = lax.axis_index("c")
        s = lax.axis_index("s")
        w = c * NSUB + s
        pltpu.sync_copy(ones_hbm, obuf)
        pltpu.sync_copy(zeros_hbm, zbuf)
        pltpu.sync_copy(dst_hbm.at[pl.ds(w * NCH, NCH)], idx_da)

        @pl.loop(0, RPS)
        def _zero(k):
            pltpu.sync_copy(zbuf, acc.at[pl.ds((s * RPS + k) * CH, CH)])

        plsc.subcore_barrier()

        @pl.loop(0, NCH - 7, step=8)
        def _edges(k):
            for b in range(8):
                pltpu.async_copy(obuf, acc.at[idx_da.at[k + b].at[0]], sem,
                                 add=True)
            for b in range(8):
                pltpu.make_async_copy(
                    obuf, acc.at[idx_da.at[k + b].at[0]], sem).wait()

        @pl.loop(NCH - NCH % 8, NCH)
        def _tail(k):
            pltpu.sync_copy(obuf, acc.at[idx_da.at[k].at[0]], add=True)

        plsc.subcore_barrier()

        @pl.loop(0, RPS)
        def _wb(k):
            r = (s * RPS + k) * CH
            pltpu.sync_copy(acc.at[pl.ds(r, CH)],
                            out_hbm.at[pl.ds(c * NP + r, CH)])

    @functools.partial(
        pl.kernel,
        out_type=jax.ShapeDtypeStruct((NCORE * NP, D), _f32),
        mesh=mesh,
        scratch_types=[
            pltpu.VMEM((NCH, 1, CH), jnp.int32),
            pltpu.VMEM((NCH, 1, CH), jnp.int32),
            pltpu.VMEM((CH, D), _f32),
            pltpu.VMEM_SHARED((NP, D), _f32),
        ],
    )
    def edge_pass(y_hbm, src_hbm, dst_hbm, zeros_hbm, out_hbm,
                  idx_sa, idx_da, gbuf, acc):
        c = lax.axis_index("c")
        s = lax.axis_index("s")
        w = c * NSUB + s
        pltpu.sync_copy(zeros_hbm, gbuf)

        @pl.loop(0, RPS)
        def _zero(k):
            pltpu.sync_copy(gbuf, acc.at[pl.ds((s * RPS + k) * CH, CH)])

        pltpu.sync_copy(src_hbm.at[pl.ds(w * NCH, NCH)], idx_sa)
        pltpu.sync_copy(dst_hbm.at[pl.ds(w * NCH, NCH)], idx_da)
        plsc.subcore_barrier()

        @pl.loop(0, NCH)
        def _edges(k):
            pltpu.sync_copy(y_hbm.at[idx_sa.at[k].at[0]], gbuf)
            pltpu.sync_copy(gbuf, acc.at[idx_da.at[k].at[0]], add=True)

        plsc.subcore_barrier()

        @pl.loop(0, RPS)
        def _wb(k):
            r = (s * RPS + k) * CH
            pltpu.sync_copy(acc.at[pl.ds(r, CH)],
                            out_hbm.at[pl.ds(c * NP + r, CH)])

    return deg_hist, edge_pass


def _dinv(deg_ref):
    d = deg_ref[0:NP, 0:1] + deg_ref[NP:2 * NP, 0:1]
    return lax.rsqrt(1.0 + d)


def _lin0_body(x_ref, w_ref, deg_ref, y_ref):
    xw = jnp.dot(x_ref[...], w_ref[...], precision=_HIGH,
                 preferred_element_type=_f32)
    y_ref[...] = _dinv(deg_ref) * xw


_lin0 = pl.pallas_call(
    _lin0_body, out_shape=jax.ShapeDtypeStruct((NP, D), _f32))


def _lin_mid_body(acc_ref, y_ref, deg_ref, b_ref, w_ref, o_ref):
    dinv = _dinv(deg_ref)
    accs = acc_ref[0:NP, :] + acc_ref[NP:2 * NP, :]
    h = jnp.maximum(dinv * (accs + y_ref[...]) + b_ref[...], 0.0)
    o_ref[...] = dinv * jnp.dot(h, w_ref[...], precision=_HIGH,
                                preferred_element_type=_f32)


_lin_mid = pl.pallas_call(
    _lin_mid_body, out_shape=jax.ShapeDtypeStruct((NP, D), _f32))


def _final_body(acc_ref, y_ref, deg_ref, b_ref, batch_ref, batch_c_ref,
                wc1_ref, bc1_ref, wc2_ref, bc2_ref, o_ref, h_ref, m_ref):
    dinv = _dinv(deg_ref)
    accs = acc_ref[0:NP, :] + acc_ref[NP:2 * NP, :]
    h = dinv * (accs + y_ref[...]) + b_ref[...]
    h_ref[...] = h

    gids = lax.broadcasted_iota(jnp.int32, (G, 1), 0)
    mask = (gids == batch_ref[...]).astype(_f32)
    cnt = jnp.sum(mask, axis=1, keepdims=True)
    sums = jnp.dot(mask, h, precision=_HIGH, preferred_element_type=_f32)
    mean = sums / jnp.maximum(cnt, 1.0)

    batch_c = batch_c_ref[...]

    def mx_step(g, _):
        cand = jnp.where(batch_c == g, h_ref[...], -jnp.inf)
        m_ref[pl.ds(g, 1), :] = jnp.max(cand, axis=0, keepdims=True)
        return 0

    lax.fori_loop(0, G, mx_step, 0)
    m = m_ref[...]
    mx = jnp.where(jnp.isfinite(m), m, 0.0)

    p = jnp.concatenate([mean, mx], axis=1)
    z = jnp.maximum(jnp.dot(p, wc1_ref[...], precision=_HIGH,
                            preferred_element_type=_f32) + bc1_ref[...], 0.0)
    o_ref[...] = jnp.dot(z, wc2_ref[...], precision=_HIGH,
                         preferred_element_type=_f32) + bc2_ref[...]


_final = pl.pallas_call(
    _final_body, out_shape=jax.ShapeDtypeStruct((G, D), _f32),
    scratch_shapes=[pltpu.VMEM((NP, D), _f32), pltpu.VMEM((G, D), _f32)])


def kernel(x, edge_index, batch, W0, b0, W1, b1, W2, b2, Wc1, bc1, Wc2, bc2):
    i32 = jnp.int32
    src_p = jnp.concatenate([edge_index[0].astype(i32),
                             jnp.zeros((EP - E,), i32)]).reshape(NW * NCH, 1, CH)
    dst_p = jnp.concatenate([edge_index[1].astype(i32),
                             jnp.full((EP - E,), N, i32)]).reshape(NW * NCH, 1, CH)
    x_p = jnp.pad(x, ((0, NP - N), (0, 0)))
    batch_p = jnp.pad(batch.astype(i32), (0, NP - N), constant_values=G)
    batch_t = batch_p.reshape(1, NP)
    batch_c = batch_p.reshape(NP, 1)
    onesD = jnp.ones((CH, D), _f32)
    zerosD = jnp.zeros((CH, D), _f32)
    wc2_p = jnp.pad(Wc2, ((0, 0), (0, D - OUT)))
    bc2_p = jnp.pad(bc2, (0, D - OUT)).reshape(1, D)

    deg_hist, edge_pass = _sc_kernels()
    deg = deg_hist(dst_p, onesD, zerosD)
    y0 = _lin0(x_p, W0, deg)
    acc0 = edge_pass(y0, src_p, dst_p, zerosD)
    y1 = _lin_mid(acc0, y0, deg, b0.reshape(1, D), W1)
    acc1 = edge_pass(y1, src_p, dst_p, zerosD)
    y2 = _lin_mid(acc1, y1, deg, b1.reshape(1, D), W2)
    acc2 = edge_pass(y2, src_p, dst_p, zerosD)
    out = _final(acc2, y2, deg, b2.reshape(1, D), batch_t, batch_c,
                 Wc1, bc1.reshape(1, D), wc2_p, bc2_p)
    return out[:, :OUT]


# speedup vs baseline: 10.3444x; 1.0394x over previous
import functools

import jax
import jax.numpy as jnp
from jax import lax
from jax.experimental import pallas as pl
from jax.experimental.pallas import tpu as pltpu
from jax.experimental.pallas import tpu_sc as plsc

N = 10000
NP = 10240
D = 128
G = 64
OUT = 10
E = 320000
CH = 128
NCORE = 2
NSUB = 16
NW = NCORE * NSUB
NCH = 79
EPW = NCH * CH
EP = EPW * NW
K0 = 94
K1 = 2 * NCH - K0
NROW = NW * NCH + 32
RPS = NP // NSUB // CH

_f32 = jnp.float32
_HIGH = jax.lax.Precision.HIGHEST


@functools.cache
def _sc_kernels():
    mesh = plsc.VectorSubcoreMesh(core_axis_name="c", subcore_axis_name="s")

    @functools.partial(
        pl.kernel,
        out_type=jax.ShapeDtypeStruct((NCORE * NP, D), _f32),
        mesh=mesh,
        scratch_types=[
            pltpu.VMEM((NCH, 1, CH), jnp.int32),
            pltpu.VMEM((CH, D), _f32),
            pltpu.VMEM((CH, D), _f32),
            pltpu.VMEM_SHARED((NP, D), _f32),
            pltpu.SemaphoreType.DMA,
        ],
    )
    def deg_hist(dst_hbm, ones_hbm, zeros_hbm, out_hbm,
                 idx_da, obuf, zbuf, acc, sem):
        c = lax.axis_index("c")
        s = lax.axis_index("s")
        w = c * NSUB + s
        pltpu.sync_copy(ones_hbm, obuf)
        pltpu.sync_copy(zeros_hbm, zbuf)
        pltpu.sync_copy(dst_hbm.at[pl.ds(w * NCH, NCH)], idx_da)

        @pl.loop(0, RPS)
        def _zero(k):
            pltpu.sync_copy(zbuf, acc.at[pl.ds((s * RPS + k) * CH, CH)])

        plsc.subcore_barrier()

        @pl.loop(0, NCH - 7, step=8)
        def _edges(k):
            for b in range(8):
                pltpu.async_copy(obuf, acc.at[idx_da.at[k + b].at[0]], sem,
                                 add=True)
            for b in range(8):
                pltpu.make_async_copy(
                    obuf, acc.at[idx_da.at[k + b].at[0]], sem).wait()

        @pl.loop(NCH - NCH % 8, NCH)
        def _tail(k):
            pltpu.sync_copy(obuf, acc.at[idx_da.at[k].at[0]], add=True)

        plsc.subcore_barrier()

        @pl.loop(0, RPS)
        def _wb(k):
            r = (s * RPS + k) * CH
            pltpu.sync_copy(acc.at[pl.ds(r, CH)],
                            out_hbm.at[pl.ds(c * NP + r, CH)])

    @functools.partial(
        pl.kernel,
        out_type=jax.ShapeDtypeStruct((NCORE * NP, D), _f32),
        mesh=mesh,
        scratch_types=[
            pltpu.VMEM((K0, 1, CH), jnp.int32),
            pltpu.VMEM((K0, 1, CH), jnp.int32),
            pltpu.VMEM((CH, D), _f32),
            pltpu.VMEM_SHARED((NP, D), _f32),
        ],
    )
    def edge_pass(y_hbm, src_hbm, dst_hbm, zeros_hbm, out_hbm,
                  idx_sa, idx_da, gbuf, acc):
        c = lax.axis_index("c")
        s = lax.axis_index("s")
        pltpu.sync_copy(zeros_hbm, gbuf)

        @pl.loop(0, RPS)
        def _zero(k):
            pltpu.sync_copy(gbuf, acc.at[pl.ds((s * RPS + k) * CH, CH)])

        base = jnp.where(c == 0, s * K0, NSUB * K0 + s * K1)
        nk = jnp.where(c == 0, K0, K1)
        pltpu.sync_copy(src_hbm.at[pl.ds(base, K0)], idx_sa)
        pltpu.sync_copy(dst_hbm.at[pl.ds(base, K0)], idx_da)
        plsc.subcore_barrier()

        @pl.loop(0, nk)
        def _edges(k):
            pltpu.sync_copy(y_hbm.at[idx_sa.at[k].at[0]], gbuf)
            pltpu.sync_copy(gbuf, acc.at[idx_da.at[k].at[0]], add=True)

        plsc.subcore_barrier()

        @pl.loop(0, RPS)
        def _wb(k):
            r = (s * RPS + k) * CH
            pltpu.sync_copy(acc.at[pl.ds(r, CH)],
                            out_hbm.at[pl.ds(c * NP + r, CH)])

    return deg_hist, edge_pass


def _dinv(deg_ref):
    d = deg_ref[0:NP, 0:1] + deg_ref[NP:2 * NP, 0:1]
    return lax.rsqrt(1.0 + d)


def _lin0_body(x_ref, w_ref, deg_ref, y_ref):
    xw = jnp.dot(x_ref[...], w_ref[...], precision=_HIGH,
                 preferred_element_type=_f32)
    y_ref[...] = _dinv(deg_ref) * xw


_lin0 = pl.pallas_call(
    _lin0_body, out_shape=jax.ShapeDtypeStruct((NP, D), _f32))


def _lin_mid_body(acc_ref, y_ref, deg_ref, b_ref, w_ref, o_ref):
    dinv = _dinv(deg_ref)
    accs = acc_ref[0:NP, :] + acc_ref[NP:2 * NP, :]
    h = jnp.maximum(dinv * (accs + y_ref[...]) + b_ref[...], 0.0)
    o_ref[...] = dinv * jnp.dot(h, w_ref[...], precision=_HIGH,
                                preferred_element_type=_f32)


_lin_mid = pl.pallas_call(
    _lin_mid_body, out_shape=jax.ShapeDtypeStruct((NP, D), _f32))


def _final_body(acc_ref, y_ref, deg_ref, b_ref, batch_ref, batch_c_ref,
                wc1_ref, bc1_ref, wc2_ref, bc2_ref, o_ref, h_ref, m_ref):
    dinv = _dinv(deg_ref)
    accs = acc_ref[0:NP, :] + acc_ref[NP:2 * NP, :]
    h = dinv * (accs + y_ref[...]) + b_ref[...]
    h_ref[...] = h

    gids = lax.broadcasted_iota(jnp.int32, (G, 1), 0)
    mask = (gids == batch_ref[...]).astype(_f32)
    cnt = jnp.sum(mask, axis=1, keepdims=True)
    sums = jnp.dot(mask, h, precision=_HIGH, preferred_element_type=_f32)
    mean = sums / jnp.maximum(cnt, 1.0)

    batch_c = batch_c_ref[...]

    def mx_step(g, _):
        cand = jnp.where(batch_c == g, h_ref[...], -jnp.inf)
        m_ref[pl.ds(g, 1), :] = jnp.max(cand, axis=0, keepdims=True)
        return 0

    lax.fori_loop(0, G, mx_step, 0)
    m = m_ref[...]
    mx = jnp.where(jnp.isfinite(m), m, 0.0)

    p = jnp.concatenate([mean, mx], axis=1)
    z = jnp.maximum(jnp.dot(p, wc1_ref[...], precision=_HIGH,
                            preferred_element_type=_f32) + bc1_ref[...], 0.0)
    o_ref[...] = jnp.dot(z, wc2_ref[...], precision=_HIGH,
                         preferred_element_type=_f32) + bc2_ref[...]


_final = pl.pallas_call(
    _final_body, out_shape=jax.ShapeDtypeStruct((G, D), _f32),
    scratch_shapes=[pltpu.VMEM((NP, D), _f32), pltpu.VMEM((G, D), _f32)])


def kernel(x, edge_index, batch, W0, b0, W1, b1, W2, b2, Wc1, bc1, Wc2, bc2):
    i32 = jnp.int32
    src_p = jnp.concatenate([edge_index[0].astype(i32),
                             jnp.zeros((NROW * CH - E,), i32)]).reshape(NROW, 1, CH)
    dst_p = jnp.concatenate([edge_index[1].astype(i32),
                             jnp.full((NROW * CH - E,), N, i32)]).reshape(NROW, 1, CH)
    x_p = jnp.pad(x, ((0, NP - N), (0, 0)))
    batch_p = jnp.pad(batch.astype(i32), (0, NP - N), constant_values=G)
    batch_t = batch_p.reshape(1, NP)
    batch_c = batch_p.reshape(NP, 1)
    onesD = jnp.ones((CH, D), _f32)
    zerosD = jnp.zeros((CH, D), _f32)
    wc2_p = jnp.pad(Wc2, ((0, 0), (0, D - OUT)))
    bc2_p = jnp.pad(bc2, (0, D - OUT)).reshape(1, D)

    deg_hist, edge_pass = _sc_kernels()
    deg = deg_hist(dst_p, onesD, zerosD)
    y0 = _lin0(x_p, W0, deg)
    acc0 = edge_pass(y0, src_p, dst_p, zerosD)
    y1 = _lin_mid(acc0, y0, deg, b0.reshape(1, D), W1)
    acc1 = edge_pass(y1, src_p, dst_p, zerosD)
    y2 = _lin_mid(acc1, y1, deg, b1.reshape(1, D), W2)
    acc2 = edge_pass(y2, src_p, dst_p, zerosD)
    out = _final(acc2, y2, deg, b2.reshape(1, D), batch_t, batch_c,
                 Wc1, bc1.reshape(1, D), wc2_p, bc2_p)
    return out[:, :OUT]


# speedup vs baseline: 11.1882x; 1.0816x over previous
import functools

import jax
import jax.numpy as jnp
from jax import lax
from jax.experimental import pallas as pl
from jax.experimental.pallas import tpu as pltpu
from jax.experimental.pallas import tpu_sc as plsc

N = 10000
NP = 10240
D = 128
G = 64
OUT = 10
E = 320000
CH = 128
NCORE = 2
NSUB = 16
NW = NCORE * NSUB
NCH = 79
EPW = NCH * CH
EP = EPW * NW
K0 = 102
K1 = 2 * NCH - K0
NROW = NW * NCH + 64
RPS = NP // NSUB // CH

_f32 = jnp.float32
_HIGH = jax.lax.Precision.HIGHEST


@functools.cache
def _sc_kernels():
    mesh = plsc.VectorSubcoreMesh(core_axis_name="c", subcore_axis_name="s")

    @functools.partial(
        pl.kernel,
        out_type=jax.ShapeDtypeStruct((NCORE * NP, D), _f32),
        mesh=mesh,
        scratch_types=[
            pltpu.VMEM((NCH, 1, CH), jnp.int32),
            pltpu.VMEM((CH, D), _f32),
            pltpu.VMEM((CH, D), _f32),
            pltpu.VMEM_SHARED((NP, D), _f32),
            pltpu.SemaphoreType.DMA,
        ],
    )
    def deg_hist(dst_hbm, ones_hbm, zeros_hbm, out_hbm,
                 idx_da, obuf, zbuf, acc, sem):
        c = lax.axis_index("c")
        s = lax.axis_index("s")
        w = c * NSUB + s
        pltpu.sync_copy(ones_hbm, obuf)
        pltpu.sync_copy(zeros_hbm, zbuf)
        pltpu.sync_copy(dst_hbm.at[pl.ds(w * NCH, NCH)], idx_da)

        @pl.loop(0, RPS)
        def _zero(k):
            pltpu.sync_copy(zbuf, acc.at[pl.ds((s * RPS + k) * CH, CH)])

        plsc.subcore_barrier()

        @pl.loop(0, NCH - 7, step=8)
        def _edges(k):
            for b in range(8):
                pltpu.async_copy(obuf, acc.at[idx_da.at[k + b].at[0]], sem,
                                 add=True)
            for b in range(8):
                pltpu.make_async_copy(
                    obuf, acc.at[idx_da.at[k + b].at[0]], sem).wait()

        @pl.loop(NCH - NCH % 8, NCH)
        def _tail(k):
            pltpu.sync_copy(obuf, acc.at[idx_da.at[k].at[0]], add=True)

        plsc.subcore_barrier()

        @pl.loop(0, RPS)
        def _wb(k):
            r = (s * RPS + k) * CH
            pltpu.sync_copy(acc.at[pl.ds(r, CH)],
                            out_hbm.at[pl.ds(c * NP + r, CH)])

    @functools.partial(
        pl.kernel,
        out_type=jax.ShapeDtypeStruct((NCORE * NP, D), _f32),
        mesh=mesh,
        scratch_types=[
            pltpu.VMEM((K0, 1, CH), jnp.int32),
            pltpu.VMEM((K0, 1, CH), jnp.int32),
            pltpu.VMEM((CH, D), _f32),
            pltpu.VMEM_SHARED((NP, D), _f32),
        ],
    )
    def edge_pass(y_hbm, src_hbm, dst_hbm, zeros_hbm, out_hbm,
                  idx_sa, idx_da, gbuf, acc):
        c = lax.axis_index("c")
        s = lax.axis_index("s")
        pltpu.sync_copy(zeros_hbm, gbuf)

        @pl.loop(0, RPS)
        def _zero(k):
            pltpu.sync_copy(gbuf, acc.at[pl.ds((s * RPS + k) * CH, CH)])

        base = jnp.where(c == 0, s * K0, NSUB * K0 + s * K1)
        nk = jnp.where(c == 0, K0, K1)
        pltpu.sync_copy(src_hbm.at[pl.ds(base, K0)], idx_sa)
        pltpu.sync_copy(dst_hbm.at[pl.ds(base, K0)], idx_da)
        plsc.subcore_barrier()

        @pl.loop(0, nk)
        def _edges(k):
            pltpu.sync_copy(y_hbm.at[idx_sa.at[k].at[0]], gbuf)
            pltpu.sync_copy(gbuf, acc.at[idx_da.at[k].at[0]], add=True)

        plsc.subcore_barrier()

        @pl.loop(0, RPS)
        def _wb(k):
            r = (s * RPS + k) * CH
            pltpu.sync_copy(acc.at[pl.ds(r, CH)],
                            out_hbm.at[pl.ds(c * NP + r, CH)])

    return deg_hist, edge_pass


def _dinv(deg_ref):
    d = deg_ref[0:NP, 0:1] + deg_ref[NP:2 * NP, 0:1]
    return lax.rsqrt(1.0 + d)


def _lin0_body(x_ref, w_ref, deg_ref, y_ref):
    xw = jnp.dot(x_ref[...], w_ref[...], precision=_HIGH,
                 preferred_element_type=_f32)
    y_ref[...] = _dinv(deg_ref) * xw


_lin0 = pl.pallas_call(
    _lin0_body, out_shape=jax.ShapeDtypeStruct((NP, D), _f32))


def _lin_mid_body(acc_ref, y_ref, deg_ref, b_ref, w_ref, o_ref):
    dinv = _dinv(deg_ref)
    accs = acc_ref[0:NP, :] + acc_ref[NP:2 * NP, :]
    h = jnp.maximum(dinv * (accs + y_ref[...]) + b_ref[...], 0.0)
    o_ref[...] = dinv * jnp.dot(h, w_ref[...], precision=_HIGH,
                                preferred_element_type=_f32)


_lin_mid = pl.pallas_call(
    _lin_mid_body, out_shape=jax.ShapeDtypeStruct((NP, D), _f32))


def _final_body(acc_ref, y_ref, deg_ref, b_ref, batch_ref, batch_c_ref,
                wc1_ref, bc1_ref, wc2_ref, bc2_ref, o_ref, h_ref, m_ref):
    dinv = _dinv(deg_ref)
    accs = acc_ref[0:NP, :] + acc_ref[NP:2 * NP, :]
    h = dinv * (accs + y_ref[...]) + b_ref[...]
    h_ref[...] = h

    gids = lax.broadcasted_iota(jnp.int32, (G, 1), 0)
    mask = (gids == batch_ref[...]).astype(_f32)
    cnt = jnp.sum(mask, axis=1, keepdims=True)
    sums = jnp.dot(mask, h, precision=_HIGH, preferred_element_type=_f32)
    mean = sums / jnp.maximum(cnt, 1.0)

    batch_c = batch_c_ref[...]

    def mx_step(g, _):
        cand = jnp.where(batch_c == g, h_ref[...], -jnp.inf)
        m_ref[pl.ds(g, 1), :] = jnp.max(cand, axis=0, keepdims=True)
        return 0

    lax.fori_loop(0, G, mx_step, 0)
    m = m_ref[...]
    mx = jnp.where(jnp.isfinite(m), m, 0.0)

    p = jnp.concatenate([mean, mx], axis=1)
    z = jnp.maximum(jnp.dot(p, wc1_ref[...], precision=_HIGH,
                            preferred_element_type=_f32) + bc1_ref[...], 0.0)
    o_ref[...] = jnp.dot(z, wc2_ref[...], precision=_HIGH,
                         preferred_element_type=_f32) + bc2_ref[...]


_final = pl.pallas_call(
    _final_body, out_shape=jax.ShapeDtypeStruct((G, D), _f32),
    scratch_shapes=[pltpu.VMEM((NP, D), _f32), pltpu.VMEM((G, D), _f32)])


def kernel(x, edge_index, batch, W0, b0, W1, b1, W2, b2, Wc1, bc1, Wc2, bc2):
    i32 = jnp.int32
    src_p = jnp.concatenate([edge_index[0].astype(i32),
                             jnp.zeros((NROW * CH - E,), i32)]).reshape(NROW, 1, CH)
    dst_p = jnp.concatenate([edge_index[1].astype(i32),
                             jnp.full((NROW * CH - E,), N, i32)]).reshape(NROW, 1, CH)
    x_p = jnp.pad(x, ((0, NP - N), (0, 0)))
    batch_p = jnp.pad(batch.astype(i32), (0, NP - N), constant_values=G)
    batch_t = batch_p.reshape(1, NP)
    batch_c = batch_p.reshape(NP, 1)
    onesD = jnp.ones((CH, D), _f32)
    zerosD = jnp.zeros((CH, D), _f32)
    wc2_p = jnp.pad(Wc2, ((0, 0), (0, D - OUT)))
    bc2_p = jnp.pad(bc2, (0, D - OUT)).reshape(1, D)

    deg_hist, edge_pass = _sc_kernels()
    deg = deg_hist(dst_p, onesD, zerosD)
    y0 = _lin0(x_p, W0, deg)
    acc0 = edge_pass(y0, src_p, dst_p, zerosD)
    y1 = _lin_mid(acc0, y0, deg, b0.reshape(1, D), W1)
    acc1 = edge_pass(y1, src_p, dst_p, zerosD)
    y2 = _lin_mid(acc1, y1, deg, b1.reshape(1, D), W2)
    acc2 = edge_pass(y2, src_p, dst_p, zerosD)
    out = _final(acc2, y2, deg, b2.reshape(1, D), batch_t, batch_c,
                 Wc1, bc1.reshape(1, D), wc2_p, bc2_p)
    return out[:, :OUT]
